# baseline (speedup 1.0000x reference)
import functools

import jax
import jax.numpy as jnp
from jax import lax
from jax.experimental import pallas as pl
from jax.experimental.pallas import tpu as pltpu
from jax.experimental.pallas import tpu_sc as plsc

NSC = 2
NTEC = 16
NW = NSC * NTEC
CHUNK = 40
NSTRIPE = 5


def _proj_body(x_ref, wa_ref, wc_ref, wn_ref, xa_ref, xc_ref, inl_ref):
    xb = x_ref[...]
    xa_ref[...] = xb @ wa_ref[...]
    xc_ref[...] = xb @ wc_ref[...]
    inl_ref[...] = jnp.maximum(xb @ wn_ref[...], 0.0)


def _node_proj(x, Wa, Wc, Wn):
    N, INP = x.shape
    H = Wa.shape[1]
    BLK = 2000
    return pl.pallas_call(
        _proj_body,
        grid=(N // BLK,),
        in_specs=[
            pl.BlockSpec((BLK, INP), lambda i: (i, 0)),
            pl.BlockSpec((INP, H), lambda i: (0, 0)),
            pl.BlockSpec((INP, H), lambda i: (0, 0)),
            pl.BlockSpec((INP, H), lambda i: (0, 0)),
        ],
        out_specs=[pl.BlockSpec((BLK, H), lambda i: (i, 0))] * 3,
        out_shape=[jax.ShapeDtypeStruct((N, H), jnp.float32)] * 3,
    )(x, Wa, Wc, Wn)


def _gather_stripe(xa, rb, xc, src3, et3, dst3, off_ch, sch):
    H = xa.shape[1]
    SE = sch * CHUNK
    BN = sch // NW
    mesh = plsc.VectorSubcoreMesh(core_axis_name="c", subcore_axis_name="s")

    @functools.partial(
        pl.kernel,
        out_type=[jax.ShapeDtypeStruct((SE, H), jnp.float32)] * 3,
        mesh=mesh,
        compiler_params=pltpu.CompilerParams(use_tc_tiling_on_sc=False),
        scratch_types=[
            pltpu.VMEM((BN, 1, CHUNK), jnp.int32),
            pltpu.VMEM((BN, 1, CHUNK), jnp.int32),
            pltpu.VMEM((BN, 1, CHUNK), jnp.int32),
            pltpu.VMEM((3, BN * CHUNK, H), jnp.float32),
            pltpu.SemaphoreType.DMA,
        ],
    )
    def k(xa_hbm, rb_hbm, xc_hbm, src_hbm, et_hbm, dst_hbm,
          ga_hbm, gb_hbm, gc_hbm,
          src_v, et_v, dst_v, bufs, sem):
        wid = lax.axis_index("c") * NTEC + lax.axis_index("s")
        lstart = wid * BN
        pltpu.sync_copy(src_hbm.at[pl.ds(off_ch + lstart, BN)], src_v)
        pltpu.sync_copy(et_hbm.at[pl.ds(off_ch + lstart, BN)], et_v)
        pltpu.sync_copy(dst_hbm.at[pl.ds(off_ch + lstart, BN)], dst_v)
        idxs = (src_v, et_v, dst_v)
        tbls = (xa_hbm, rb_hbm, xc_hbm)
        outs = (ga_hbm, gb_hbm, gc_hbm)

        descs = []
        for j in range(BN):
            for t in range(3):
                descs.append(pltpu.async_copy(
                    tbls[t].at[idxs[t].at[j, 0]],
                    bufs.at[t, pl.ds(j * CHUNK, CHUNK)], sem))
        for d in descs:
            d.wait()
        for t in range(3):
            pltpu.sync_copy(bufs.at[t],
                            outs[t].at[pl.ds(lstart * CHUNK, BN * CHUNK)])

    return k(xa, rb, xc, src3, et3, dst3)


def _edge_chain(ga, gb, gc, egi3, GT, W3, WA2, A2S):
    E, H = ga.shape
    B = GT.shape[1]
    TILE = egi3.shape[2]

    def body(ga_ref, gb_ref, gc_ref, gi_ref, gt_ref, w3_ref, wa_ref, a2_ref,
             out_ref):
        me0 = jnp.maximum(ga_ref[...] + gb_ref[...] + gc_ref[...], 0.0)
        gid = gi_ref[0, 0, :]
        onehot = (gid[:, None] == lax.broadcasted_iota(
            jnp.int32, (TILE, B), 1)).astype(jnp.float32)
        Gg = onehot @ gt_ref[0]
        T0g = onehot @ gt_ref[1]
        T1g = onehot @ gt_ref[2]
        h0 = jnp.maximum(me0 @ w3_ref[0] + Gg, 0.0)
        att0 = jax.nn.sigmoid(h0 @ a2_ref[0])
        me0p = me0 * att0
        me1 = jnp.maximum(me0p @ w3_ref[1], 0.0)
        h1 = jnp.maximum(me1 @ wa_ref[0] + T0g, 0.0)
        att1 = jax.nn.sigmoid(h1 @ a2_ref[1])
        me1p = me1 * att1
        me2 = jnp.maximum(me1p @ w3_ref[2], 0.0)
        h2 = jnp.maximum(me2 @ wa_ref[1] + T1g, 0.0)
        att2 = jax.nn.sigmoid(h2 @ a2_ref[2])
        me2p = me2 * att2
        pad = jnp.zeros((TILE, 128 - 3 * H), jnp.float32)
        out_ref[...] = jnp.concatenate([me0p, me1p, me2p, pad], axis=1)

    return pl.pallas_call(
        body,
        grid=(E // TILE,),
        in_specs=[
            pl.BlockSpec((TILE, H), lambda i: (i, 0)),
            pl.BlockSpec((TILE, H), lambda i: (i, 0)),
            pl.BlockSpec((TILE, H), lambda i: (i, 0)),
            pl.BlockSpec((1, 1, TILE), lambda i: (i, 0, 0)),
            pl.BlockSpec(GT.shape, lambda i: (0, 0, 0)),
            pl.BlockSpec(W3.shape, lambda i: (0, 0, 0)),
            pl.BlockSpec(WA2.shape, lambda i: (0, 0, 0)),
            pl.BlockSpec(A2S.shape, lambda i: (0, 0, 0)),
        ],
        out_specs=pl.BlockSpec((TILE, 128), lambda i: (i, 0)),
        out_shape=jax.ShapeDtypeStruct((E, 128), jnp.float32),
    )(ga, gb, gc, egi3, GT, W3, WA2, A2S)


def _scatter_partials(meps, dst3, zrows, N, sch):
    F = meps[0].shape[1]
    BN = sch // NW
    SEG = 5
    NSEG = BN // SEG
    RMAIN = -(-(N // NTEC) // 8) * 8
    RLAST = N - (NTEC - 1) * RMAIN
    mesh = plsc.VectorSubcoreMesh(core_axis_name="c", subcore_axis_name="s")

    @functools.partial(
        pl.kernel,
        out_type=jax.ShapeDtypeStruct((NSC, N, F), jnp.float32),
        mesh=mesh,
        scratch_types=[
            pltpu.VMEM((BN, 1, CHUNK), jnp.int32),
            pltpu.VMEM((SEG * CHUNK, F), jnp.float32),
            pltpu.VMEM_SHARED((N, F), jnp.float32),
            pltpu.SemaphoreType.DMA,
        ],
    )
    def k(m0, m1, m2, m3, m4, dsti_hbm, z_hbm, part_hbm,
          dst_v, buf, shared, sem):
        mrefs = (m0, m1, m2, m3, m4)
        c = lax.axis_index("c")
        s = lax.axis_index("s")
        wid = c * NTEC + s
        lstart = wid * BN

        @pl.when(s < NTEC - 1)
        def _():
            pltpu.sync_copy(z_hbm.at[pl.ds(s * RMAIN, RMAIN)],
                            shared.at[pl.ds(s * RMAIN, RMAIN)])

        @pl.when(s == NTEC - 1)
        def _():
            pltpu.sync_copy(z_hbm.at[pl.ds((NTEC - 1) * RMAIN, RLAST)],
                            shared.at[pl.ds((NTEC - 1) * RMAIN, RLAST)])

        plsc.subcore_barrier()

        pend = []
        for st in range(len(meps)):
            mref = mrefs[st]
            pltpu.sync_copy(dsti_hbm.at[pl.ds(st * sch + lstart, BN)],
                            dst_v)
            r0 = lstart * CHUNK
            for seg in range(NSEG):
                for d in pend:
                    d.wait()
                pltpu.sync_copy(
                    mref.at[pl.ds(r0 + seg * SEG * CHUNK, SEG * CHUNK)],
                    buf)
                pend = [
                    pltpu.async_copy(
                        buf.at[pl.ds(j * CHUNK, CHUNK)],
                        shared.at[dst_v.at[seg * SEG + j, 0]],
                        sem, add=True)
                    for j in range(SEG)
                ]
            for d in pend:
                d.wait()
            pend = []

        plsc.subcore_barrier()

        @pl.when(s < NTEC - 1)
        def _():
            pltpu.sync_copy(shared.at[pl.ds(s * RMAIN, RMAIN)],
                            part_hbm.at[c, pl.ds(s * RMAIN, RMAIN)])

        @pl.when(s == NTEC - 1)
        def _():
            pltpu.sync_copy(shared.at[pl.ds((NTEC - 1) * RMAIN, RLAST)],
                            part_hbm.at[c, pl.ds((NTEC - 1) * RMAIN, RLAST)])

    return k(*meps, dst3, zrows)


def _node_final(part, inl, S3, Wn2, Wc3, Wo_sum, bo2, sn2, tn2, Rg, w12, c12):
    N, H = inl.shape
    B = Rg.shape[0]

    def body(part_ref, inl_ref, s3_ref, wn_ref, wc_ref, wo_ref, bo_ref,
             sn_ref, tn_ref, rg_ref, w12_ref, c12_ref, out_ref):
        p = part_ref[0] + part_ref[1]
        agg0 = p @ s3_ref[0]
        agg1 = p @ s3_ref[1]
        agg2 = p @ s3_ref[2]
        inl = inl_ref[...]
        mn1 = jnp.maximum((inl + agg0) @ wn_ref[0], 0.0)
        mn2 = jnp.maximum((mn1 + agg1) @ wn_ref[1], 0.0)
        aggC = agg2 @ wc_ref[0] + mn2 @ wc_ref[1] + inl @ wc_ref[2]
        nh = jnp.maximum(
            jnp.maximum(aggC, 0.0) @ wo_ref[...] + bo_ref[...], 0.0)
        iota_n = lax.broadcasted_iota(jnp.int32, (B, N), 1)
        oh_s = (iota_n == sn_ref[...]).astype(jnp.float32)
        oh_t = (iota_n == tn_ref[...]).astype(jnp.float32)
        src_e = oh_s @ nh
        tgt_e = oh_t @ nh
        conv = jnp.tanh(src_e + rg_ref[...] - tgt_e)
        out_ref[...] = jnp.sum(
            conv * w12_ref[...], axis=1, keepdims=True) + c12_ref[...]

    return pl.pallas_call(
        body,
        out_shape=jax.ShapeDtypeStruct((B, 1), jnp.float32),
    )(part, inl, S3, Wn2, Wc3, Wo_sum, bo2, sn2, tn2, Rg, w12, c12)


def kernel(x, edge_index, edge_type, graph_rel, node_graph_ids,
           edge_graph_ids, source_nodes, target_nodes, R, W_i_node, W_i_edge,
           A1, A2, Wh_node0, Wh_node1, Wh_edge0, Wh_edge1, Att1_0, Att2_0,
           Att1_1, Att2_1, W_comm, W_o, b_o, W1, b1, W2, b2):
    N, INP = x.shape
    E = edge_index.shape[1]
    B = graph_rel.shape[0]
    REL = R.shape[1]
    H = W_i_node.shape[1]

    Wa = W_i_edge[:INP]
    Wb = W_i_edge[INP:INP + REL]
    Wc = W_i_edge[INP + REL:]

    xa, xc, inl = _node_proj(x, Wa, Wc, W_i_node)

    Rb = R @ Wb
    Rg = jnp.take(R, graph_rel, axis=0)
    g_pg = jnp.take(inl, source_nodes, axis=0) + Rg \
        - jnp.take(inl, target_nodes, axis=0)
    GT = jnp.stack([g_pg @ A1[:H], Rg @ Att1_0[H:], Rg @ Att1_1[H:]])
    W3 = jnp.stack([A1[H:], Wh_edge0, Wh_edge1])
    WA2 = jnp.stack([Att1_0[:H], Att1_1[:H]])
    ones_row = jnp.ones((1, H), jnp.float32)
    A2S = jnp.stack([A2 @ ones_row, Att2_0 @ ones_row, Att2_1 @ ones_row])

    NCH = E // CHUNK
    SCH = NCH // NSTRIPE
    SE = SCH * CHUNK

    def chunked(a):
        return a.reshape(NCH, 1, CHUNK)

    src3 = chunked(edge_index[0])
    et3 = chunked(edge_type)
    dst3 = chunked(edge_index[1])

    TILE = 4000
    meps = []
    for st in range(NSTRIPE):
        ga, gb, gc = _gather_stripe(xa, Rb, xc, src3, et3, dst3,
                                    st * SCH, SCH)
        egi3 = edge_graph_ids[st * SE:(st + 1) * SE].reshape(
            SE // TILE, 1, TILE)
        meps.append(_edge_chain(ga, gb, gc, egi3, GT, W3, WA2, A2S))

    zrows = jnp.zeros((N, 128), jnp.float32)
    part = _scatter_partials(meps, dst3, zrows, N, SCH)

    eye = jnp.eye(H, dtype=jnp.float32)
    zz = jnp.zeros((H, H), jnp.float32)
    zp = jnp.zeros((128 - 3 * H, H), jnp.float32)
    S3 = jnp.stack([
        jnp.concatenate([eye, zz, zz, zp], axis=0),
        jnp.concatenate([zz, eye, zz, zp], axis=0),
        jnp.concatenate([zz, zz, eye, zp], axis=0),
    ])
    Wn2 = jnp.stack([Wh_node0, Wh_node1])
    Wc3 = jnp.stack([W_comm[:H], W_comm[H:2 * H], W_comm[2 * H:]])
    Wo_sum = W_o[:H] + W_o[H:]
    bo2 = b_o.reshape(1, H)
    sn2 = source_nodes.reshape(B, 1)
    tn2 = target_nodes.reshape(B, 1)
    w12 = (W1 @ W2).reshape(1, H)
    c12 = (b1 @ W2 + b2).reshape(1, 1)

    return _node_final(part, inl, S3, Wn2, Wc3, Wo_sum, bo2, sn2, tn2,
                       Rg, w12, c12)


# speedup vs baseline: 1.1053x; 1.1053x over previous
import functools

import jax
import jax.numpy as jnp
from jax import lax
from jax.experimental import pallas as pl
from jax.experimental.pallas import tpu as pltpu
from jax.experimental.pallas import tpu_sc as plsc

NSC = 2
NTEC = 16
NW = NSC * NTEC
CHUNK = 128
NSTRIPE = 5


def _proj_body(x_ref, wa_ref, wc_ref, wn_ref, xa_ref, xc_ref, inl_ref):
    xb = x_ref[...]
    xa_ref[...] = xb @ wa_ref[...]
    xc_ref[...] = xb @ wc_ref[...]
    inl_ref[...] = jnp.maximum(xb @ wn_ref[...], 0.0)


def _node_proj(x, Wa, Wc, Wn):
    N, INP = x.shape
    H = Wa.shape[1]
    BLK = 2000
    return pl.pallas_call(
        _proj_body,
        grid=(N // BLK,),
        in_specs=[
            pl.BlockSpec((BLK, INP), lambda i: (i, 0)),
            pl.BlockSpec((INP, H), lambda i: (0, 0)),
            pl.BlockSpec((INP, H), lambda i: (0, 0)),
            pl.BlockSpec((INP, H), lambda i: (0, 0)),
        ],
        out_specs=[pl.BlockSpec((BLK, H), lambda i: (i, 0))] * 3,
        out_shape=[jax.ShapeDtypeStruct((N, H), jnp.float32)] * 3,
    )(x, Wa, Wc, Wn)


def _gather_stripe(xa, rb, xc, src3, et3, dst3, off_ch, sch):
    H = xa.shape[1]
    SE = sch * CHUNK
    BN, RM = sch // NW, sch % NW
    SLAB = BN + 1
    mesh = plsc.VectorSubcoreMesh(core_axis_name="c", subcore_axis_name="s")

    @functools.partial(
        pl.kernel,
        out_type=[jax.ShapeDtypeStruct((SE, H), jnp.float32)] * 3,
        mesh=mesh,
        compiler_params=pltpu.CompilerParams(use_tc_tiling_on_sc=False),
        scratch_types=[
            pltpu.VMEM((SLAB, 1, CHUNK), jnp.int32),
            pltpu.VMEM((SLAB, 1, CHUNK), jnp.int32),
            pltpu.VMEM((SLAB, 1, CHUNK), jnp.int32),
            pltpu.VMEM((3, SLAB * CHUNK, H), jnp.float32),
            pltpu.SemaphoreType.DMA,
        ],
    )
    def k(xa_hbm, rb_hbm, xc_hbm, src_hbm, et_hbm, dst_hbm,
          ga_hbm, gb_hbm, gc_hbm,
          src_v, et_v, dst_v, bufs, sem):
        wid = lax.axis_index("c") * NTEC + lax.axis_index("s")
        lstart = wid * BN + jnp.minimum(wid, RM)
        w8 = wid < RM
        pltpu.sync_copy(src_hbm.at[pl.ds(off_ch + lstart, SLAB)], src_v)
        pltpu.sync_copy(et_hbm.at[pl.ds(off_ch + lstart, SLAB)], et_v)
        pltpu.sync_copy(dst_hbm.at[pl.ds(off_ch + lstart, SLAB)], dst_v)
        idxs = (src_v, et_v, dst_v)
        tbls = (xa_hbm, rb_hbm, xc_hbm)
        outs = (ga_hbm, gb_hbm, gc_hbm)

        descs = []
        for j in range(BN):
            for t in range(3):
                descs.append(pltpu.async_copy(
                    tbls[t].at[idxs[t].at[j, 0]],
                    bufs.at[t, pl.ds(j * CHUNK, CHUNK)], sem))
        for d in descs:
            d.wait()

        @pl.when(w8)
        def _():
            last = [pltpu.async_copy(
                tbls[t].at[idxs[t].at[BN, 0]],
                bufs.at[t, pl.ds(BN * CHUNK, CHUNK)], sem)
                for t in range(3)]
            for d in last:
                d.wait()
            for t in range(3):
                pltpu.sync_copy(
                    bufs.at[t],
                    outs[t].at[pl.ds(lstart * CHUNK, SLAB * CHUNK)])

        @pl.when(jnp.logical_not(w8))
        def _():
            for t in range(3):
                pltpu.sync_copy(
                    bufs.at[t, pl.ds(0, BN * CHUNK)],
                    outs[t].at[pl.ds(lstart * CHUNK, BN * CHUNK)])

    return k(xa, rb, xc, src3, et3, dst3)


def _edge_chain(ga, gb, gc, egi3, GT, W3, WA2, A2S):
    E, H = ga.shape
    B = GT.shape[1]
    TILE = egi3.shape[2]

    def body(ga_ref, gb_ref, gc_ref, gi_ref, gt_ref, w3_ref, wa_ref, a2_ref,
             out_ref):
        me0 = jnp.maximum(ga_ref[...] + gb_ref[...] + gc_ref[...], 0.0)
        gid = gi_ref[0, 0, :]
        onehot = (gid[:, None] == lax.broadcasted_iota(
            jnp.int32, (TILE, B), 1)).astype(jnp.float32)
        Gg = onehot @ gt_ref[0]
        T0g = onehot @ gt_ref[1]
        T1g = onehot @ gt_ref[2]
        h0 = jnp.maximum(me0 @ w3_ref[0] + Gg, 0.0)
        att0 = jax.nn.sigmoid(h0 @ a2_ref[0])
        me0p = me0 * att0
        me1 = jnp.maximum(me0p @ w3_ref[1], 0.0)
        h1 = jnp.maximum(me1 @ wa_ref[0] + T0g, 0.0)
        att1 = jax.nn.sigmoid(h1 @ a2_ref[1])
        me1p = me1 * att1
        me2 = jnp.maximum(me1p @ w3_ref[2], 0.0)
        h2 = jnp.maximum(me2 @ wa_ref[1] + T1g, 0.0)
        att2 = jax.nn.sigmoid(h2 @ a2_ref[2])
        me2p = me2 * att2
        pad = jnp.zeros((TILE, 128 - 3 * H), jnp.float32)
        out_ref[...] = jnp.concatenate([me0p, me1p, me2p, pad], axis=1)

    return pl.pallas_call(
        body,
        grid=(E // TILE,),
        in_specs=[
            pl.BlockSpec((TILE, H), lambda i: (i, 0)),
            pl.BlockSpec((TILE, H), lambda i: (i, 0)),
            pl.BlockSpec((TILE, H), lambda i: (i, 0)),
            pl.BlockSpec((1, 1, TILE), lambda i: (i, 0, 0)),
            pl.BlockSpec(GT.shape, lambda i: (0, 0, 0)),
            pl.BlockSpec(W3.shape, lambda i: (0, 0, 0)),
            pl.BlockSpec(WA2.shape, lambda i: (0, 0, 0)),
            pl.BlockSpec(A2S.shape, lambda i: (0, 0, 0)),
        ],
        out_specs=pl.BlockSpec((TILE, 128), lambda i: (i, 0)),
        out_shape=jax.ShapeDtypeStruct((E, 128), jnp.float32),
    )(ga, gb, gc, egi3, GT, W3, WA2, A2S)


def _scatter_partials(meps, dst3, zrows, N, sch):
    F = meps[0].shape[1]
    NCH = sch * len(meps)
    BNCH, REM = NCH // NW, NCH % NW
    SLAB = BNCH + 1
    RMAIN = -(-(N // NTEC) // 8) * 8
    RLAST = N - (NTEC - 1) * RMAIN
    mesh = plsc.VectorSubcoreMesh(core_axis_name="c", subcore_axis_name="s")

    @functools.partial(
        pl.kernel,
        out_type=jax.ShapeDtypeStruct((NSC, N, F), jnp.float32),
        mesh=mesh,
        scratch_types=[
            pltpu.VMEM((SLAB, 1, CHUNK), jnp.int32),
            pltpu.VMEM((CHUNK, F), jnp.float32),
            pltpu.VMEM_SHARED((N, F), jnp.float32),
            pltpu.SemaphoreType.DMA,
        ],
    )
    def k(m0, m1, m2, m3, m4, dsti_hbm, z_hbm, part_hbm,
          dst_v, buf, shared, sem):
        mrefs = (m0, m1, m2, m3, m4)
        c = lax.axis_index("c")
        s = lax.axis_index("s")
        wid = c * NTEC + s
        start = wid * BNCH + jnp.minimum(wid, REM)
        nch = BNCH + (wid < REM).astype(jnp.int32)
        end = start + nch
        pltpu.sync_copy(dsti_hbm.at[pl.ds(start, SLAB)], dst_v)

        @pl.when(s < NTEC - 1)
        def _():
            pltpu.sync_copy(z_hbm.at[pl.ds(s * RMAIN, RMAIN)],
                            shared.at[pl.ds(s * RMAIN, RMAIN)])

        @pl.when(s == NTEC - 1)
        def _():
            pltpu.sync_copy(z_hbm.at[pl.ds((NTEC - 1) * RMAIN, RLAST)],
                            shared.at[pl.ds((NTEC - 1) * RMAIN, RLAST)])

        plsc.subcore_barrier()

        for st in range(len(meps)):
            mref = mrefs[st]
            lo = jnp.maximum(start, st * sch)
            hi = jnp.minimum(end, (st + 1) * sch)

            def body(g, carry, mref=mref, st=st):
                r = (g - st * sch) * CHUNK
                pltpu.sync_copy(mref.at[pl.ds(r, CHUNK)], buf)
                pltpu.sync_copy(buf, shared.at[dst_v.at[g - start, 0]],
                                add=True)
                return carry

            lax.fori_loop(lo, hi, body, 0)

        plsc.subcore_barrier()

        @pl.when(s < NTEC - 1)
        def _():
            pltpu.sync_copy(shared.at[pl.ds(s * RMAIN, RMAIN)],
                            part_hbm.at[c, pl.ds(s * RMAIN, RMAIN)])

        @pl.when(s == NTEC - 1)
        def _():
            pltpu.sync_copy(shared.at[pl.ds((NTEC - 1) * RMAIN, RLAST)],
                            part_hbm.at[c, pl.ds((NTEC - 1) * RMAIN, RLAST)])

    return k(*meps, dst3, zrows)


def _node_final(part, inl, S3, Wn2, Wc3, Wo_sum, bo2, sn2, tn2, Rg, w12, c12):
    N, H = inl.shape
    B = Rg.shape[0]

    def body(part_ref, inl_ref, s3_ref, wn_ref, wc_ref, wo_ref, bo_ref,
             sn_ref, tn_ref, rg_ref, w12_ref, c12_ref, out_ref):
        p = part_ref[0] + part_ref[1]
        agg0 = p @ s3_ref[0]
        agg1 = p @ s3_ref[1]
        agg2 = p @ s3_ref[2]
        inl = inl_ref[...]
        mn1 = jnp.maximum((inl + agg0) @ wn_ref[0], 0.0)
        mn2 = jnp.maximum((mn1 + agg1) @ wn_ref[1], 0.0)
        aggC = agg2 @ wc_ref[0] + mn2 @ wc_ref[1] + inl @ wc_ref[2]
        nh = jnp.maximum(
            jnp.maximum(aggC, 0.0) @ wo_ref[...] + bo_ref[...], 0.0)
        iota_n = lax.broadcasted_iota(jnp.int32, (B, N), 1)
        oh_s = (iota_n == sn_ref[...]).astype(jnp.float32)
        oh_t = (iota_n == tn_ref[...]).astype(jnp.float32)
        src_e = oh_s @ nh
        tgt_e = oh_t @ nh
        conv = jnp.tanh(src_e + rg_ref[...] - tgt_e)
        out_ref[...] = jnp.sum(
            conv * w12_ref[...], axis=1, keepdims=True) + c12_ref[...]

    return pl.pallas_call(
        body,
        out_shape=jax.ShapeDtypeStruct((B, 1), jnp.float32),
    )(part, inl, S3, Wn2, Wc3, Wo_sum, bo2, sn2, tn2, Rg, w12, c12)


def kernel(x, edge_index, edge_type, graph_rel, node_graph_ids,
           edge_graph_ids, source_nodes, target_nodes, R, W_i_node, W_i_edge,
           A1, A2, Wh_node0, Wh_node1, Wh_edge0, Wh_edge1, Att1_0, Att2_0,
           Att1_1, Att2_1, W_comm, W_o, b_o, W1, b1, W2, b2):
    N, INP = x.shape
    E = edge_index.shape[1]
    B = graph_rel.shape[0]
    REL = R.shape[1]
    H = W_i_node.shape[1]

    Wa = W_i_edge[:INP]
    Wb = W_i_edge[INP:INP + REL]
    Wc = W_i_edge[INP + REL:]

    xa, xc, inl = _node_proj(x, Wa, Wc, W_i_node)

    Rb = R @ Wb
    Rg = jnp.take(R, graph_rel, axis=0)
    g_pg = jnp.take(inl, source_nodes, axis=0) + Rg \
        - jnp.take(inl, target_nodes, axis=0)
    GT = jnp.stack([g_pg @ A1[:H], Rg @ Att1_0[H:], Rg @ Att1_1[H:]])
    W3 = jnp.stack([A1[H:], Wh_edge0, Wh_edge1])
    WA2 = jnp.stack([Att1_0[:H], Att1_1[:H]])
    ones_row = jnp.ones((1, H), jnp.float32)
    A2S = jnp.stack([A2 @ ones_row, Att2_0 @ ones_row, Att2_1 @ ones_row])

    NCH = E // CHUNK
    SCH = NCH // NSTRIPE
    SE = SCH * CHUNK
    PAD = (NCH + NW) * CHUNK - E

    def chunked(a):
        return jnp.pad(a, (0, PAD)).reshape(NCH + NW, 1, CHUNK)

    src3 = chunked(edge_index[0])
    et3 = chunked(edge_type)
    dst3 = chunked(edge_index[1])

    TILE = 4000
    meps = []
    for st in range(NSTRIPE):
        ga, gb, gc = _gather_stripe(xa, Rb, xc, src3, et3, dst3,
                                    st * SCH, SCH)
        egi3 = edge_graph_ids[st * SE:(st + 1) * SE].reshape(
            SE // TILE, 1, TILE)
        meps.append(_edge_chain(ga, gb, gc, egi3, GT, W3, WA2, A2S))

    zrows = jnp.zeros((N, 128), jnp.float32)
    part = _scatter_partials(meps, dst3, zrows, N, SCH)

    eye = jnp.eye(H, dtype=jnp.float32)
    zz = jnp.zeros((H, H), jnp.float32)
    zp = jnp.zeros((128 - 3 * H, H), jnp.float32)
    S3 = jnp.stack([
        jnp.concatenate([eye, zz, zz, zp], axis=0),
        jnp.concatenate([zz, eye, zz, zp], axis=0),
        jnp.concatenate([zz, zz, eye, zp], axis=0),
    ])
    Wn2 = jnp.stack([Wh_node0, Wh_node1])
    Wc3 = jnp.stack([W_comm[:H], W_comm[H:2 * H], W_comm[2 * H:]])
    Wo_sum = W_o[:H] + W_o[H:]
    bo2 = b_o.reshape(1, H)
    sn2 = source_nodes.reshape(B, 1)
    tn2 = target_nodes.reshape(B, 1)
    w12 = (W1 @ W2).reshape(1, H)
    c12 = (b1 @ W2 + b2).reshape(1, 1)

    return _node_final(part, inl, S3, Wn2, Wc3, Wo_sum, bo2, sn2, tn2,
                       Rg, w12, c12)
